# Initial kernel scaffold; baseline (speedup 1.0000x reference)
import jax
import jax.numpy as jnp
from jax.experimental import pallas as pl


def kernel(x, pos, batch, params):
    raise NotImplementedError("write your pallas kernel here")


# speedup vs baseline: 6.7290x; 6.7290x over previous
import functools

import jax
import jax.numpy as jnp
from jax import lax
from jax.experimental import pallas as pl
from jax.experimental.pallas import tpu as pltpu
from jax.experimental.pallas import tpu_sc as plsc

_F32 = jnp.float32
_BF16 = jnp.bfloat16


def _bdot(a, b):
    return lax.dot_general(a.astype(_BF16), b.astype(_BF16),
                           (((1,), (0,)), ((), ())),
                           preferred_element_type=_F32)


def _fps(px, py, pz, n):
    R = px.shape[0]
    N = R * 128
    M = n // 128

    def body(px_ref, py_ref, pz_ref, ox_ref, oy_ref, oz_ref):
        pxv, pyv, pzv = px_ref[...], py_ref[...], pz_ref[...]
        ii = (lax.broadcasted_iota(jnp.int32, (R, 128), 0) * 128
              + lax.broadcasted_iota(jnp.int32, (R, 128), 1))
        oi = (lax.broadcasted_iota(jnp.int32, (M, 128), 0) * 128
              + lax.broadcasted_iota(jnp.int32, (M, 128), 1))

        def sel_coords(am):
            oh = ii == am
            sx = jnp.sum(jnp.where(oh, pxv, 0.0))
            sy = jnp.sum(jnp.where(oh, pyv, 0.0))
            sz = jnp.sum(jnp.where(oh, pzv, 0.0))
            return sx, sy, sz

        sx, sy, sz = sel_coords(jnp.int32(0))
        dx = pxv - sx
        dy = pyv - sy
        dz = pzv - sz
        dmin0 = dx * dx + dy * dy + dz * dz
        ox0 = jnp.where(oi == 0, sx, jnp.float32(0.0))
        oy0 = jnp.where(oi == 0, sy, jnp.float32(0.0))
        oz0 = jnp.where(oi == 0, sz, jnp.float32(0.0))

        def step(t, c):
            dmin, ox, oy, oz = c
            m = jnp.max(dmin)
            am = jnp.min(jnp.where(dmin == m, ii, N))
            sx, sy, sz = sel_coords(am)
            ox = jnp.where(oi == t, sx, ox)
            oy = jnp.where(oi == t, sy, oy)
            oz = jnp.where(oi == t, sz, oz)
            dx = pxv - sx
            dy = pyv - sy
            dz = pzv - sz
            d = dx * dx + dy * dy + dz * dz
            return jnp.minimum(dmin, d), ox, oy, oz

        _, ox, oy, oz = lax.fori_loop(1, n, step, (dmin0, ox0, oy0, oz0))
        ox_ref[...] = ox
        oy_ref[...] = oy
        oz_ref[...] = oz

    return pl.pallas_call(
        body,
        out_shape=[jax.ShapeDtypeStruct((M, 128), _F32)] * 3,
    )(px, py, pz)


def _knn(dst, srcT, k, Bd, exact_d):
    Nd = dst.shape[0]
    Ns = srcT.shape[1]
    grid = Nd // Bd

    def body(dst_ref, srcT_ref, oi_ref, od_ref):
        dstv = dst_ref[...]
        s = srcT_ref[...]
        sn = jnp.sum(s * s, axis=0, keepdims=True)
        dn = jnp.sum(dstv * dstv, axis=1, keepdims=True)
        D = jnp.maximum(dn + sn - 2.0 * _bdot(dstv, s), 0.0)
        ji = lax.broadcasted_iota(jnp.int32, (Bd, Ns), 1)
        ki = lax.broadcasted_iota(jnp.int32, (Bd, k), 1)
        BIG = jnp.float32(3.0e38)
        sx, sy, sz = s[0:1, :], s[1:2, :], s[2:3, :]
        dxv, dyv, dzv = dstv[:, 0:1], dstv[:, 1:2], dstv[:, 2:3]

        def step(t, c):
            D, oi, od = c
            m = jnp.min(D, axis=1, keepdims=True)
            am = jnp.min(jnp.where(D == m, ji, Ns), axis=1, keepdims=True)
            onb = ji == am
            if exact_d:
                gx = jnp.sum(jnp.where(onb, sx, 0.0), axis=1, keepdims=True)
                gy = jnp.sum(jnp.where(onb, sy, 0.0), axis=1, keepdims=True)
                gz = jnp.sum(jnp.where(onb, sz, 0.0), axis=1, keepdims=True)
                ex = dxv - gx
                ey = dyv - gy
                ez = dzv - gz
                dval = jnp.sqrt(jnp.maximum(ex * ex + ey * ey + ez * ez, 0.0))
            else:
                dval = jnp.sqrt(jnp.maximum(m, 0.0))
            oi = jnp.where(ki == t, am, oi)
            od = jnp.where(ki == t, dval, od)
            D = jnp.where(onb, BIG, D)
            return D, oi, od

        _, oi, od = lax.fori_loop(
            0, k, step,
            (D, jnp.zeros((Bd, k), jnp.int32), jnp.zeros((Bd, k), _F32)))
        oi_ref[...] = oi
        od_ref[...] = od

    return pl.pallas_call(
        body,
        grid=(grid,),
        in_specs=[pl.BlockSpec((Bd, 8), lambda i: (i, 0)),
                  pl.BlockSpec((8, Ns), lambda i: (0, 0))],
        out_specs=[pl.BlockSpec((Bd, k), lambda i: (i, 0)),
                   pl.BlockSpec((Bd, k), lambda i: (i, 0))],
        out_shape=[jax.ShapeDtypeStruct((Nd, k), jnp.int32),
                   jax.ShapeDtypeStruct((Nd, k), _F32)],
    )(dst, srcT)


def _sc_gather(table, idx):
    B = idx.shape[0]
    D = table.shape[1]
    NW = 32
    bpw = B // NW
    C = min(128, bpw)
    nch = bpw // C
    mesh = plsc.VectorSubcoreMesh(core_axis_name="c", subcore_axis_name="s")

    @functools.partial(
        pl.kernel,
        out_type=jax.ShapeDtypeStruct((B, D), _F32),
        mesh=mesh,
        scratch_types=[pltpu.VMEM((C,), jnp.int32),
                       pltpu.VMEM((C, D), _F32),
                       pltpu.SemaphoreType.DMA],
    )
    def k(table_hbm, idx_hbm, out_hbm, idx_v, rows_v, sem):
        wid = lax.axis_index("s") * 2 + lax.axis_index("c")
        base = wid * bpw

        def chunk(c, carry):
            off = base + c * C
            pltpu.sync_copy(idx_hbm.at[pl.ds(off, C)], idx_v)
            pltpu.async_copy(table_hbm.at[idx_v], rows_v, sem).wait()
            pltpu.sync_copy(rows_v, out_hbm.at[pl.ds(off, C)])
            return carry

        lax.fori_loop(0, nch, chunk, 0)

    return k(table, idx)


def _pconv1(G, dstp, w1, b1, w2, b2, v1, c1, v2, c2):
    Bd, k = 128, 32
    grid = 4096 // Bd

    def body(G_ref, d_ref, w1_ref, b1_ref, w2_ref, b2_ref, v1_ref, c1_ref,
             v2_ref, c2_ref, x1_ref):
        dstv = d_ref[...]
        G3 = G_ref[...].reshape(Bd, k, 128)
        xj = G3[:, :, 0:4]
        rel = G3[:, :, 4:7] - dstv[:, None, 0:3]
        msg = jnp.concatenate([xj, rel], axis=2).reshape(Bd * k, 7)
        h1 = jnp.maximum(_bdot(msg, w1_ref[...]) + b1_ref[...], 0.0)
        h2 = _bdot(h1, w2_ref[...]) + b2_ref[...]
        hmax = jnp.max(h2.reshape(Bd, k, h2.shape[1]), axis=1)
        g1 = jnp.maximum(_bdot(hmax, v1_ref[...]) + c1_ref[...], 0.0)
        x1_ref[...] = _bdot(g1, v2_ref[...]) + c2_ref[...]

    full = lambda a: pl.BlockSpec(a.shape, lambda i: tuple(0 for _ in a.shape))
    return pl.pallas_call(
        body,
        grid=(grid,),
        in_specs=[pl.BlockSpec((Bd * k, 128), lambda i: (i, 0)),
                  pl.BlockSpec((Bd, 8), lambda i: (i, 0)),
                  full(w1), full(b1), full(w2), full(b2), full(v1), full(c1),
                  full(v2), full(c2)],
        out_specs=pl.BlockSpec((Bd, 128), lambda i: (i, 0)),
        out_shape=jax.ShapeDtypeStruct((4096, 128), _F32),
    )(G, dstp, w1, b1, w2, b2, v1, c1, v2, c2)


def _pconv2(G, dstp, w1, b1, w2, b2, v1, c1, v2, c2):
    Bd, k = 64, 64
    grid = 1024 // Bd

    def body(G_ref, d_ref, w1_ref, b1_ref, w2_ref, b2_ref, v1_ref, c1_ref,
             v2_ref, c2_ref, x2_ref):
        dstv = d_ref[...]
        G3 = G_ref[...].reshape(Bd, k, 256)
        xj = G3[:, :, 0:128]
        rel = G3[:, :, 128:131] - dstv[:, None, 0:3]
        msg = jnp.concatenate([xj, rel], axis=2).reshape(Bd * k, 131)
        h1 = jnp.maximum(_bdot(msg, w1_ref[...]) + b1_ref[...], 0.0)
        h2 = _bdot(h1, w2_ref[...]) + b2_ref[...]
        hmax = jnp.max(h2.reshape(Bd, k, h2.shape[1]), axis=1)
        g1 = jnp.maximum(_bdot(hmax, v1_ref[...]) + c1_ref[...], 0.0)
        x2_ref[...] = _bdot(g1, v2_ref[...]) + c2_ref[...]

    full = lambda a: pl.BlockSpec(a.shape, lambda i: tuple(0 for _ in a.shape))
    return pl.pallas_call(
        body,
        grid=(grid,),
        in_specs=[pl.BlockSpec((Bd * k, 256), lambda i: (i, 0)),
                  pl.BlockSpec((Bd, 8), lambda i: (i, 0)),
                  full(w1), full(b1), full(w2), full(b2), full(v1), full(c1),
                  full(v2), full(c2)],
        out_specs=pl.BlockSpec((Bd, 256), lambda i: (i, 0)),
        out_shape=jax.ShapeDtypeStruct((1024, 256), _F32),
    )(G, dstp, w1, b1, w2, b2, v1, c1, v2, c2)


def _fp2_kernel(xc, Xg, d3, wa, wb, b1, w2, b2):
    n = xc.shape[0]

    def body(xc_ref, xg_ref, d_ref, wa_ref, wb_ref, b1_ref, w2_ref, b2_ref,
             o_ref):
        d = d_ref[...]
        w = 1.0 / (d + 1e-8)
        w = w / jnp.sum(w, axis=1, keepdims=True)
        x0 = xg_ref[0:n, :]
        x1_ = xg_ref[n:2 * n, :]
        x2_ = xg_ref[2 * n:3 * n, :]
        xint = w[:, 0:1] * x0 + w[:, 1:2] * x1_ + w[:, 2:3] * x2_
        h = jnp.maximum(_bdot(xc_ref[...], wa_ref[...])
                        + _bdot(xint, wb_ref[...]) + b1_ref[...], 0.0)
        o_ref[...] = _bdot(h, w2_ref[...]) + b2_ref[...]

    return pl.pallas_call(
        body,
        out_shape=jax.ShapeDtypeStruct((n, w2.shape[1]), _F32),
    )(xc, Xg, d3, wa, wb, b1, w2, b2)


def _fp1_kernel(xc, Xg, d3, wa, wb, b1, w2, b2, wsem, bsem, wemb, bemb):
    n = xc.shape[0]

    def body(xc_ref, xg_ref, d_ref, wa_ref, wb_ref, b1_ref, w2_ref, b2_ref,
             ws_ref, bs_ref, we_ref, be_ref, sem_ref, emb_ref):
        d = d_ref[...]
        w = 1.0 / (d + 1e-8)
        w = w / jnp.sum(w, axis=1, keepdims=True)
        x0 = xg_ref[0:n, :]
        x1_ = xg_ref[n:2 * n, :]
        x2_ = xg_ref[2 * n:3 * n, :]
        xint = w[:, 0:1] * x0 + w[:, 1:2] * x1_ + w[:, 2:3] * x2_
        h = jnp.maximum(_bdot(xc_ref[...], wa_ref[...])
                        + _bdot(xint, wb_ref[...]) + b1_ref[...], 0.0)
        xfp = _bdot(h, w2_ref[...]) + b2_ref[...]
        sem_ref[...] = _bdot(xfp, ws_ref[...]) + bs_ref[...]
        emb_ref[...] = _bdot(xfp, we_ref[...]) + be_ref[...]

    return pl.pallas_call(
        body,
        out_shape=[jax.ShapeDtypeStruct((n, 8), _F32),
                   jax.ShapeDtypeStruct((n, wemb.shape[1]), _F32)],
    )(xc, Xg, d3, wa, wb, b1, w2, b2, wsem, bsem, wemb, bemb)


def _row(v):
    return v.reshape(1, -1)


def kernel(x, pos, batch, params):
    N = pos.shape[0]
    posT = pos.T
    srcT_pos = jnp.concatenate([posT, jnp.zeros((5, N), _F32)], axis=0)

    px = posT[0].reshape(N // 128, 128)
    py = posT[1].reshape(N // 128, 128)
    pz = posT[2].reshape(N // 128, 128)

    p1x, p1y, p1z = _fps(px, py, pz, N // 2)
    p1T = jnp.stack([p1x.reshape(-1), p1y.reshape(-1), p1z.reshape(-1)])
    srcT_p1 = jnp.concatenate([p1T, jnp.zeros((5, N // 2), _F32)], axis=0)
    p1p = srcT_p1.T

    p2x, p2y, p2z = _fps(p1x, p1y, p1z, N // 8)
    p2T = jnp.stack([p2x.reshape(-1), p2y.reshape(-1), p2z.reshape(-1)])
    p2p = jnp.concatenate([p2T, jnp.zeros((5, N // 8), _F32)], axis=0).T

    prm = params
    (w1, b1), (w2, b2) = prm['sa1_local']
    (v1, c1), (v2, c2) = prm['sa1_global']
    (nw1, nb1), (nw2, nb2) = prm['sa2_local']
    (u1, e1), (u2, e2) = prm['sa2_global']

    table1 = jnp.concatenate([x, pos, jnp.zeros((N, 121), _F32)], axis=1)
    nbr1, _ = _knn(p1p, srcT_pos, 32, 128, False)
    G1 = _sc_gather(table1, nbr1.reshape(-1))
    x1 = _pconv1(G1, p1p, w1, _row(b1), w2, _row(b2), v1, _row(c1),
                 v2, _row(c2))

    table2 = jnp.concatenate(
        [x1, p1T.T, jnp.zeros((N // 2, 125), _F32)], axis=1)
    nbr2, _ = _knn(p2p, srcT_p1, 64, 128, False)
    G2 = _sc_gather(table2, nbr2.reshape(-1))
    x2 = _pconv2(G2, p2p, nw1, _row(nb1), nw2, _row(nb2), u1, _row(e1),
                 u2, _row(e2))

    (fw1, fb1), (fw2, fb2) = prm['fp2']
    fp2i, fp2d = _knn(p2p, srcT_p1, 3, 128, True)
    F2 = _sc_gather(x1, fp2i.T.reshape(-1))
    xfp2 = _fp2_kernel(x2, F2, fp2d, fw1[0:256], fw1[256:384], _row(fb1),
                       fw2, _row(fb2))

    (gw1, gb1), (gw2, gb2) = prm['fp1']
    gwb = jnp.concatenate(
        [gw1[256:260], jnp.zeros((124, gw1.shape[1]), _F32)], 0)
    ws, bs = prm['sem']
    wsp = jnp.concatenate([ws, jnp.zeros((ws.shape[0], 5), _F32)], 1)
    bsp = jnp.concatenate([bs, jnp.zeros((5,), _F32)])
    we, be = prm['inst']
    fp1i, fp1d = _knn(p2p, srcT_pos, 3, 128, True)
    F1 = _sc_gather(table1, fp1i.T.reshape(-1))
    semp, emb = _fp1_kernel(xfp2, F1, fp1d, gw1[0:256], gwb, _row(gb1),
                            gw2, _row(gb2), wsp, _row(bsp), we, _row(be))
    return (semp[:, 0:3], emb)


# speedup vs baseline: 7.0642x; 1.0498x over previous
import functools

import jax
import jax.numpy as jnp
from jax import lax
from jax.experimental import pallas as pl
from jax.experimental.pallas import tpu as pltpu
from jax.experimental.pallas import tpu_sc as plsc

_F32 = jnp.float32
_BF16 = jnp.bfloat16


def _bdot(a, b):
    return lax.dot_general(a.astype(_BF16), b.astype(_BF16),
                           (((1,), (0,)), ((), ())),
                           preferred_element_type=_F32)


def _fps(px, py, pz, n):
    R = px.shape[0]
    N = R * 128
    M = n // 128

    def body(px_ref, py_ref, pz_ref, ox_ref, oy_ref, oz_ref):
        pxv, pyv, pzv = px_ref[...], py_ref[...], pz_ref[...]
        ii = (lax.broadcasted_iota(jnp.int32, (R, 128), 0) * 128
              + lax.broadcasted_iota(jnp.int32, (R, 128), 1))

        def sel_coords(am):
            oh = ii == am
            spx = jnp.sum(jnp.where(oh, pxv, 0.0), axis=0, keepdims=True)
            spy = jnp.sum(jnp.where(oh, pyv, 0.0), axis=0, keepdims=True)
            spz = jnp.sum(jnp.where(oh, pzv, 0.0), axis=0, keepdims=True)
            sp = jnp.concatenate([spx, spy, spz], axis=0)
            ssum = jnp.sum(sp, axis=1, keepdims=True)
            return ssum[0:1, 0:1], ssum[1:2, 0:1], ssum[2:3, 0:1]

        def emit(t, sx, sy, sz):
            ox_ref[pl.ds(t, 1), :] = sx
            oy_ref[pl.ds(t, 1), :] = sy
            oz_ref[pl.ds(t, 1), :] = sz

        sx, sy, sz = sel_coords(jnp.int32(0))
        emit(0, sx, sy, sz)
        dx = pxv - sx
        dy = pyv - sy
        dz = pzv - sz
        dmin0 = dx * dx + dy * dy + dz * dz

        def step(t, dmin):
            m = jnp.max(dmin)
            am = jnp.min(jnp.where(dmin == m, ii, N))
            sx, sy, sz = sel_coords(am)
            emit(t, sx, sy, sz)
            dx = pxv - sx
            dy = pyv - sy
            dz = pzv - sz
            d = dx * dx + dy * dy + dz * dz
            return jnp.minimum(dmin, d)

        lax.fori_loop(1, n, step, dmin0)

    outs = pl.pallas_call(
        body,
        out_shape=[jax.ShapeDtypeStruct((n, 1), _F32)] * 3,
    )(px, py, pz)
    return [o.reshape(M, 128) for o in outs]


def _knn(dst, srcT, k, Bd, exact_d):
    Nd = dst.shape[0]
    Ns = srcT.shape[1]
    grid = Nd // Bd

    def body(dst_ref, srcT_ref, oi_ref, od_ref):
        dstv = dst_ref[...]
        s = srcT_ref[...]
        sn = jnp.sum(s * s, axis=0, keepdims=True)
        dn = jnp.sum(dstv * dstv, axis=1, keepdims=True)
        D = jnp.maximum(dn + sn - 2.0 * _bdot(dstv, s), 0.0)
        ji = lax.broadcasted_iota(jnp.int32, (Bd, Ns), 1)
        ki = lax.broadcasted_iota(jnp.int32, (Bd, k), 1)
        BIG = jnp.float32(3.0e38)
        sx, sy, sz = s[0:1, :], s[1:2, :], s[2:3, :]
        dxv, dyv, dzv = dstv[:, 0:1], dstv[:, 1:2], dstv[:, 2:3]

        def step(exact, t, c):
            D, oi, od = c
            m = jnp.min(D, axis=1, keepdims=True)
            am = jnp.min(jnp.where(D == m, ji, Ns), axis=1, keepdims=True)
            onb = ji == am
            if exact:
                gx = jnp.sum(jnp.where(onb, sx, 0.0), axis=1, keepdims=True)
                gy = jnp.sum(jnp.where(onb, sy, 0.0), axis=1, keepdims=True)
                gz = jnp.sum(jnp.where(onb, sz, 0.0), axis=1, keepdims=True)
                ex = dxv - gx
                ey = dyv - gy
                ez = dzv - gz
                dval = jnp.sqrt(jnp.maximum(ex * ex + ey * ey + ez * ez, 0.0))
            else:
                dval = jnp.sqrt(jnp.maximum(m, 0.0))
            oi = jnp.where(ki == t, am, oi)
            od = jnp.where(ki == t, dval, od)
            D = jnp.where(onb, BIG, D)
            return D, oi, od

        c = (D, jnp.zeros((Bd, k), jnp.int32), jnp.zeros((Bd, k), _F32))
        u = min(3, k) if exact_d else 0
        for t in range(u):
            c = step(True, t, c)
        _, oi, od = lax.fori_loop(u, k, functools.partial(step, False), c)
        oi_ref[...] = oi
        od_ref[...] = od

    return pl.pallas_call(
        body,
        grid=(grid,),
        in_specs=[pl.BlockSpec((Bd, 8), lambda i: (i, 0)),
                  pl.BlockSpec((8, Ns), lambda i: (0, 0))],
        out_specs=[pl.BlockSpec((Bd, k), lambda i: (i, 0)),
                   pl.BlockSpec((Bd, k), lambda i: (i, 0))],
        out_shape=[jax.ShapeDtypeStruct((Nd, k), jnp.int32),
                   jax.ShapeDtypeStruct((Nd, k), _F32)],
    )(dst, srcT)


def _sc_gather(table, idx):
    B = idx.shape[0]
    D = table.shape[1]
    NW = 32
    bpw = B // NW
    C = min(128, bpw)
    nch = bpw // C
    mesh = plsc.VectorSubcoreMesh(core_axis_name="c", subcore_axis_name="s")

    @functools.partial(
        pl.kernel,
        out_type=jax.ShapeDtypeStruct((B, D), _F32),
        mesh=mesh,
        scratch_types=[pltpu.VMEM((C,), jnp.int32),
                       pltpu.VMEM((C, D), _F32),
                       pltpu.SemaphoreType.DMA],
    )
    def k(table_hbm, idx_hbm, out_hbm, idx_v, rows_v, sem):
        wid = lax.axis_index("s") * 2 + lax.axis_index("c")
        base = wid * bpw

        def chunk(c, carry):
            off = base + c * C
            pltpu.sync_copy(idx_hbm.at[pl.ds(off, C)], idx_v)
            pltpu.async_copy(table_hbm.at[idx_v], rows_v, sem).wait()
            pltpu.sync_copy(rows_v, out_hbm.at[pl.ds(off, C)])
            return carry

        lax.fori_loop(0, nch, chunk, 0)

    return k(table, idx)


def _pconv1(G, dstp, w1, b1, w2, b2, v1, c1, v2, c2):
    Bd, k = 128, 32
    grid = 4096 // Bd

    def body(G_ref, d_ref, w1_ref, b1_ref, w2_ref, b2_ref, v1_ref, c1_ref,
             v2_ref, c2_ref, x1_ref):
        dstv = d_ref[...]
        G3 = G_ref[...].reshape(Bd, k, 128)
        xj = G3[:, :, 0:4]
        rel = G3[:, :, 4:7] - dstv[:, None, 0:3]
        msg = jnp.concatenate([xj, rel], axis=2).reshape(Bd * k, 7)
        h1 = jnp.maximum(_bdot(msg, w1_ref[...]) + b1_ref[...], 0.0)
        h2 = _bdot(h1, w2_ref[...]) + b2_ref[...]
        hmax = jnp.max(h2.reshape(Bd, k, h2.shape[1]), axis=1)
        g1 = jnp.maximum(_bdot(hmax, v1_ref[...]) + c1_ref[...], 0.0)
        x1_ref[...] = _bdot(g1, v2_ref[...]) + c2_ref[...]

    full = lambda a: pl.BlockSpec(a.shape, lambda i: tuple(0 for _ in a.shape))
    return pl.pallas_call(
        body,
        grid=(grid,),
        in_specs=[pl.BlockSpec((Bd * k, 128), lambda i: (i, 0)),
                  pl.BlockSpec((Bd, 8), lambda i: (i, 0)),
                  full(w1), full(b1), full(w2), full(b2), full(v1), full(c1),
                  full(v2), full(c2)],
        out_specs=pl.BlockSpec((Bd, 128), lambda i: (i, 0)),
        out_shape=jax.ShapeDtypeStruct((4096, 128), _F32),
    )(G, dstp, w1, b1, w2, b2, v1, c1, v2, c2)


def _pconv2(G, dstp, w1, b1, w2, b2, v1, c1, v2, c2):
    Bd, k = 64, 64
    grid = 1024 // Bd

    def body(G_ref, d_ref, w1_ref, b1_ref, w2_ref, b2_ref, v1_ref, c1_ref,
             v2_ref, c2_ref, x2_ref):
        dstv = d_ref[...]
        G3 = G_ref[...].reshape(Bd, k, 256)
        xj = G3[:, :, 0:128]
        rel = G3[:, :, 128:131] - dstv[:, None, 0:3]
        msg = jnp.concatenate([xj, rel], axis=2).reshape(Bd * k, 131)
        h1 = jnp.maximum(_bdot(msg, w1_ref[...]) + b1_ref[...], 0.0)
        h2 = _bdot(h1, w2_ref[...]) + b2_ref[...]
        hmax = jnp.max(h2.reshape(Bd, k, h2.shape[1]), axis=1)
        g1 = jnp.maximum(_bdot(hmax, v1_ref[...]) + c1_ref[...], 0.0)
        x2_ref[...] = _bdot(g1, v2_ref[...]) + c2_ref[...]

    full = lambda a: pl.BlockSpec(a.shape, lambda i: tuple(0 for _ in a.shape))
    return pl.pallas_call(
        body,
        grid=(grid,),
        in_specs=[pl.BlockSpec((Bd * k, 256), lambda i: (i, 0)),
                  pl.BlockSpec((Bd, 8), lambda i: (i, 0)),
                  full(w1), full(b1), full(w2), full(b2), full(v1), full(c1),
                  full(v2), full(c2)],
        out_specs=pl.BlockSpec((Bd, 256), lambda i: (i, 0)),
        out_shape=jax.ShapeDtypeStruct((1024, 256), _F32),
    )(G, dstp, w1, b1, w2, b2, v1, c1, v2, c2)


def _fp2_kernel(xc, Xg, d3, wa, wb, b1, w2, b2):
    n = xc.shape[0]

    def body(xc_ref, xg_ref, d_ref, wa_ref, wb_ref, b1_ref, w2_ref, b2_ref,
             o_ref):
        d = d_ref[...]
        w = 1.0 / (d + 1e-8)
        w = w / jnp.sum(w, axis=1, keepdims=True)
        x0 = xg_ref[0:n, :]
        x1_ = xg_ref[n:2 * n, :]
        x2_ = xg_ref[2 * n:3 * n, :]
        xint = w[:, 0:1] * x0 + w[:, 1:2] * x1_ + w[:, 2:3] * x2_
        h = jnp.maximum(_bdot(xc_ref[...], wa_ref[...])
                        + _bdot(xint, wb_ref[...]) + b1_ref[...], 0.0)
        o_ref[...] = _bdot(h, w2_ref[...]) + b2_ref[...]

    return pl.pallas_call(
        body,
        out_shape=jax.ShapeDtypeStruct((n, w2.shape[1]), _F32),
    )(xc, Xg, d3, wa, wb, b1, w2, b2)


def _fp1_kernel(xc, Xg, d3, wa, wb, b1, w2, b2, wsem, bsem, wemb, bemb):
    n = xc.shape[0]

    def body(xc_ref, xg_ref, d_ref, wa_ref, wb_ref, b1_ref, w2_ref, b2_ref,
             ws_ref, bs_ref, we_ref, be_ref, sem_ref, emb_ref):
        d = d_ref[...]
        w = 1.0 / (d + 1e-8)
        w = w / jnp.sum(w, axis=1, keepdims=True)
        x0 = xg_ref[0:n, :]
        x1_ = xg_ref[n:2 * n, :]
        x2_ = xg_ref[2 * n:3 * n, :]
        xint = w[:, 0:1] * x0 + w[:, 1:2] * x1_ + w[:, 2:3] * x2_
        h = jnp.maximum(_bdot(xc_ref[...], wa_ref[...])
                        + _bdot(xint, wb_ref[...]) + b1_ref[...], 0.0)
        xfp = _bdot(h, w2_ref[...]) + b2_ref[...]
        sem_ref[...] = _bdot(xfp, ws_ref[...]) + bs_ref[...]
        emb_ref[...] = _bdot(xfp, we_ref[...]) + be_ref[...]

    return pl.pallas_call(
        body,
        out_shape=[jax.ShapeDtypeStruct((n, 8), _F32),
                   jax.ShapeDtypeStruct((n, wemb.shape[1]), _F32)],
    )(xc, Xg, d3, wa, wb, b1, w2, b2, wsem, bsem, wemb, bemb)


def _row(v):
    return v.reshape(1, -1)


def kernel(x, pos, batch, params):
    N = pos.shape[0]
    posT = pos.T
    srcT_pos = jnp.concatenate([posT, jnp.zeros((5, N), _F32)], axis=0)

    px = posT[0].reshape(N // 128, 128)
    py = posT[1].reshape(N // 128, 128)
    pz = posT[2].reshape(N // 128, 128)

    p1x, p1y, p1z = _fps(px, py, pz, N // 2)
    p1T = jnp.stack([p1x.reshape(-1), p1y.reshape(-1), p1z.reshape(-1)])
    srcT_p1 = jnp.concatenate([p1T, jnp.zeros((5, N // 2), _F32)], axis=0)
    p1p = srcT_p1.T

    p2x, p2y, p2z = _fps(p1x, p1y, p1z, N // 8)
    p2T = jnp.stack([p2x.reshape(-1), p2y.reshape(-1), p2z.reshape(-1)])
    p2p = jnp.concatenate([p2T, jnp.zeros((5, N // 8), _F32)], axis=0).T

    prm = params
    (w1, b1), (w2, b2) = prm['sa1_local']
    (v1, c1), (v2, c2) = prm['sa1_global']
    (nw1, nb1), (nw2, nb2) = prm['sa2_local']
    (u1, e1), (u2, e2) = prm['sa2_global']

    table1 = jnp.concatenate([x, pos, jnp.zeros((N, 121), _F32)], axis=1)
    nbr1, _ = _knn(p1p, srcT_pos, 32, 128, False)
    G1 = _sc_gather(table1, nbr1.reshape(-1))
    x1 = _pconv1(G1, p1p, w1, _row(b1), w2, _row(b2), v1, _row(c1),
                 v2, _row(c2))

    table2 = jnp.concatenate(
        [x1, p1T.T, jnp.zeros((N // 2, 125), _F32)], axis=1)
    nbr2, nbr2d = _knn(p2p, srcT_p1, 64, 128, True)
    G2 = _sc_gather(table2, nbr2.reshape(-1))
    x2 = _pconv2(G2, p2p, nw1, _row(nb1), nw2, _row(nb2), u1, _row(e1),
                 u2, _row(e2))

    (fw1, fb1), (fw2, fb2) = prm['fp2']
    fp2i, fp2d = nbr2[:, 0:3], nbr2d[:, 0:3]
    F2 = _sc_gather(x1, fp2i.T.reshape(-1))
    xfp2 = _fp2_kernel(x2, F2, fp2d, fw1[0:256], fw1[256:384], _row(fb1),
                       fw2, _row(fb2))

    (gw1, gb1), (gw2, gb2) = prm['fp1']
    gwb = jnp.concatenate(
        [gw1[256:260], jnp.zeros((124, gw1.shape[1]), _F32)], 0)
    ws, bs = prm['sem']
    wsp = jnp.concatenate([ws, jnp.zeros((ws.shape[0], 5), _F32)], 1)
    bsp = jnp.concatenate([bs, jnp.zeros((5,), _F32)])
    we, be = prm['inst']
    fp1i, fp1d = _knn(p2p, srcT_pos, 3, 128, True)
    F1 = _sc_gather(table1, fp1i.T.reshape(-1))
    semp, emb = _fp1_kernel(xfp2, F1, fp1d, gw1[0:256], gwb, _row(gb1),
                            gw2, _row(gb2), wsp, _row(bsp), we, _row(be))
    return (semp[:, 0:3], emb)
